# baseline (speedup 1.0000x reference)
import functools

import jax
import jax.numpy as jnp
from jax import lax
from jax.experimental import pallas as pl
from jax.experimental.pallas import tpu as pltpu
from jax.experimental.pallas import tpu_sc as plsc

EMB_D = 32
BATCH_N = 16384
NUM_ROWS = 1000000
PACK = 128 // EMB_D
NUM_CORES = 2
NUM_SUBCORES = 16
NW = NUM_CORES * NUM_SUBCORES
CHUNK = 128
B_PER_W = BATCH_N // NW
NCHUNK = B_PER_W // CHUNK
LANES = 16
NGROUP = CHUNK // LANES

_mesh = plsc.VectorSubcoreMesh(core_axis_name="c", subcore_axis_name="s")


@functools.partial(
    pl.kernel,
    mesh=_mesh,
    out_type=jax.ShapeDtypeStruct((BATCH_N // PACK, 128), jnp.float32),
    compiler_params=pltpu.CompilerParams(use_tc_tiling_on_sc=True),
    scratch_types=[
        pltpu.VMEM((B_PER_W + LANES,), jnp.int32),
        pltpu.VMEM((B_PER_W,), jnp.int32),
        pltpu.VMEM((CHUNK, 128), jnp.float32),
        pltpu.VMEM((CHUNK, 128), jnp.float32),
        pltpu.VMEM((CHUNK, 128), jnp.float32),
        pltpu.VMEM((CHUNK, 128), jnp.float32),
        pltpu.VMEM((B_PER_W // PACK, 128), jnp.float32),
        pltpu.SemaphoreType.DMA,
    ],
)
def _gather_rows(idx_hbm, table_hbm, out_hbm, idx_v, fidx_v, r0_v, r1_v, r2_v,
                 r3_v, out_v, sem):
    rows_bufs = [r0_v, r1_v, r2_v, r3_v]
    wid = lax.axis_index("s") * NUM_CORES + lax.axis_index("c")
    base = wid * B_PER_W
    pltpu.sync_copy(idx_hbm.at[pl.ds(base, B_PER_W)],
                    idx_v.at[pl.ds(0, B_PER_W)])

    copies = []
    for c in range(NCHUNK):
        for g in range(NGROUP):
            off = c * CHUNK + g * LANES
            fidx_v[pl.ds(off, LANES)] = lax.shift_right_logical(
                idx_v[pl.ds(off, LANES)], 2)
        copies.append(
            pltpu.async_copy(
                table_hbm.at[fidx_v.at[pl.ds(c * CHUNK, CHUNK)]],
                rows_bufs[c], sem))

    for c in range(NCHUNK):
        copies[c].wait()
        rows_c = rows_bufs[c]

        def body(r, _, c=c, rows_c=rows_c):
            br = c * CHUNK + r
            idx_r = idx_v[pl.ds(br, LANES)][0]
            off = lax.shift_left(idx_r & (PACK - 1), 5)
            ocol = lax.shift_left(br & (PACK - 1), 5)
            orow = lax.shift_right_logical(br, 2)
            for h in range(EMB_D // LANES):
                out_v[orow, pl.ds(ocol + h * LANES, LANES)] = (
                    rows_c[r, pl.ds(off + h * LANES, LANES)])
            return ()

        lax.fori_loop(0, CHUNK, body, ())
    pltpu.sync_copy(
        out_v, out_hbm.at[pl.ds(wid * (B_PER_W // PACK), B_PER_W // PACK)])


def kernel(elements, entity_embeddings):
    table4 = entity_embeddings.reshape(NUM_ROWS // PACK, 128)
    out4 = _gather_rows(elements.astype(jnp.int32), table4)
    return out4.reshape(BATCH_N, EMB_D)


# speedup vs baseline: 1.6749x; 1.6749x over previous
import functools

import jax
import jax.numpy as jnp
from jax import lax
from jax.experimental import pallas as pl
from jax.experimental.pallas import tpu as pltpu
from jax.experimental.pallas import tpu_sc as plsc

EMB_D = 32
BATCH_N = 16384
NUM_CORES = 2
NUM_SUBCORES = 16
NW = NUM_CORES * NUM_SUBCORES
B_PER_W = BATCH_N // NW
LANES = 16

_mesh = plsc.VectorSubcoreMesh(core_axis_name="c", subcore_axis_name="s")


@functools.partial(
    pl.kernel,
    mesh=_mesh,
    out_type=jax.ShapeDtypeStruct((BATCH_N, EMB_D), jnp.float32),
    compiler_params=pltpu.CompilerParams(use_tc_tiling_on_sc=True),
    scratch_types=[
        pltpu.VMEM((B_PER_W + LANES,), jnp.int32),
        pltpu.VMEM((B_PER_W, EMB_D), jnp.float32),
        pltpu.SemaphoreType.DMA,
    ],
)
def _gather_rows(idx_hbm, table_hbm, out_hbm, idx_v, out_v, sem):
    wid = lax.axis_index("s") * NUM_CORES + lax.axis_index("c")
    base = wid * B_PER_W
    pltpu.sync_copy(idx_hbm.at[pl.ds(base, B_PER_W)],
                    idx_v.at[pl.ds(0, B_PER_W)])

    def body(r, _):
        idx_r = idx_v[pl.ds(r, LANES)][0]
        pltpu.async_copy(table_hbm.at[idx_r], out_v.at[r], sem)
        return ()

    lax.fori_loop(0, B_PER_W, body, ())
    pltpu.make_async_copy(
        table_hbm.at[pl.ds(0, B_PER_W)], out_v, sem).wait()
    pltpu.sync_copy(out_v, out_hbm.at[pl.ds(base, B_PER_W)])


def kernel(elements, entity_embeddings):
    return _gather_rows(elements.astype(jnp.int32), entity_embeddings)
